# baseline (speedup 1.0000x reference)
import functools

import jax
import jax.numpy as jnp
from jax import lax
from jax.experimental import pallas as pl
from jax.experimental.pallas import tpu as pltpu
from jax.experimental.pallas import tpu_sc as plsc

NUM_MOL = 100000
NW = 32
MPW = 3136
MPW_LAST = NUM_MOL - (NW - 1) * MPW
ACC_PAD = MPW
W = 16384
WE = W - 16
CS = 2048


def _sc_body(n_atoms, vals_hbm, idx_hbm, out_hbm,
             vbufa, ibufa, vbufb, ibufb, acc, cilist, csbuf, sema, semb):
    c = lax.axis_index("c")
    s = lax.axis_index("s")
    w = s * 2 + c
    mw0 = w * MPW
    mpw_w = jnp.minimum(mw0 + MPW, NUM_MOL) - mw0

    iota0 = lax.iota(jnp.int32, 16)
    ncs = n_atoms // CS
    ncs_pad = ((ncs + 15) // 16) * 16

    off = w * 97

    def bld(i, _):
        slot = i * 16 + iota0
        pos = slot + off
        pos = jnp.where(pos >= ncs, pos - ncs, pos)
        pos = jnp.where(slot >= ncs, 0, pos)
        cilist[pl.ds(i * 16, 16)] = pos * CS
        return 0

    lax.fori_loop(0, ncs_pad // 16, bld, 0)
    pltpu.async_copy(idx_hbm.at[cilist], csbuf, sema).wait()

    t0v = jnp.full((16,), mw0, dtype=jnp.int32)
    t1v = jnp.full((16,), mw0 + mpw_w, dtype=jnp.int32)

    def cnt(i, carry):
        c0, c1 = carry
        slot = i * 16 + iota0
        valid = slot < ncs
        cs_v = csbuf[pl.ds(i * 16, 16)]
        c0 = c0 + plsc.all_reduce_population_count((cs_v < t0v) & valid)
        c1 = c1 + plsc.all_reduce_population_count((cs_v < t1v) & valid)
        return c0, c1

    zc = jnp.zeros((16,), jnp.int32)
    j0v, j1v = lax.fori_loop(0, ncs_pad // 16, cnt, (zc, zc))
    a0 = jnp.maximum(0, (j0v[0] - 1) * CS)
    a1 = jnp.minimum(n_atoms, j1v[0] * CS)

    sent = jnp.full((16,), NUM_MOL, dtype=jnp.int32)
    ibufa[pl.ds(W, 16)] = sent
    ibufb[pl.ds(W, 16)] = sent

    def zbody(i, _):
        acc[pl.ds(i * 16, 16)] = jnp.zeros((16,), jnp.float32)
        return 0

    lax.fori_loop(0, ACC_PAD // 16, zbody, 0)

    iota = lax.iota(jnp.int32, 16)
    m_end15 = iota == 15
    m_low15 = iota < 15
    mpw_u = jnp.full((16,), mpw_w, dtype=jnp.int32)

    nwin = jnp.maximum((a1 - a0 + WE - 1) // WE, 1)
    nfull = nwin - 1

    def issue(kk, vbuf, ibuf, sem):
        p0 = a0 + kk * WE
        st = jnp.minimum(p0, n_atoms - W)
        st = pl.multiple_of(st, 16)
        pltpu.async_copy(vals_hbm.at[pl.ds(st, W)], vbuf.at[pl.ds(0, W)], sem)
        pltpu.async_copy(idx_hbm.at[pl.ds(st, W)], ibuf.at[pl.ds(0, W)], sem)

    def wait(vbuf, ibuf, sem):
        pltpu.make_async_copy(vals_hbm.at[pl.ds(0, W)],
                              vbuf.at[pl.ds(0, W)], sem).wait()
        pltpu.make_async_copy(idx_hbm.at[pl.ds(0, W)],
                              ibuf.at[pl.ds(0, W)], sem).wait()

    issue(0, vbufa, ibufa, sema)

    def body_at(vbuf, ibuf, base, j):
        o = base + j * 16
        v = vbuf[pl.ds(o, 16)]
        ic = ibuf[pl.ds(o, 16)]
        inx = ibuf[pl.ds(o + 1, 16)]
        cum = plsc.cumsum(v)
        li = ic - mw0
        ln = inx - mw0
        chg = ic != inx
        in_i = plsc.bitcast(li, jnp.uint32) < plsc.bitcast(mpw_u, jnp.uint32)
        in_n = plsc.bitcast(ln, jnp.uint32) < plsc.bitcast(mpw_u, jnp.uint32)
        mend = (chg | m_end15) & in_i
        msub = chg & m_low15 & in_n
        plsc.addupdate_scatter(acc, [li], cum, mask=mend)
        plsc.addupdate_scatter(acc, [ln], -cum, mask=msub)

    def lean_loop(vbuf, ibuf):
        @plsc.parallel_loop(0, WE // 16, unroll=16)
        def _(j):
            body_at(vbuf, ibuf, 0, j)

    def lean_w(k, _unused):
        @pl.when(k % 2 == 0)
        def _():
            wait(vbufa, ibufa, sema)
            issue(k + 1, vbufb, ibufb, semb)
            lean_loop(vbufa, ibufa)

        @pl.when(k % 2 == 1)
        def _():
            wait(vbufb, ibufb, semb)
            issue(k + 1, vbufa, ibufa, sema)
            lean_loop(vbufb, ibufb)

        return 0

    lax.fori_loop(0, nfull, lean_w, 0)

    p0t = a0 + nfull * WE
    stt = jnp.minimum(p0t, n_atoms - W)
    base = pl.multiple_of(p0t - stt, 16)
    nblk = (a1 - p0t + 15) // 16

    def tail_loop(vbuf, ibuf):
        @plsc.parallel_loop(0, nblk, unroll=4)
        def _(j):
            body_at(vbuf, ibuf, base, j)

    @pl.when(nfull % 2 == 0)
    def _():
        wait(vbufa, ibufa, sema)
        tail_loop(vbufa, ibufa)

    @pl.when(nfull % 2 == 1)
    def _():
        wait(vbufb, ibufb, semb)
        tail_loop(vbufb, ibufb)

    @pl.when(w < NW - 1)
    def _():
        pltpu.sync_copy(acc.at[pl.ds(0, MPW)], out_hbm.at[pl.ds(mw0, MPW)])

    @pl.when(w == NW - 1)
    def _():
        pltpu.sync_copy(acc.at[pl.ds(0, MPW_LAST)],
                        out_hbm.at[pl.ds(mw0, MPW_LAST)])


@jax.jit
def kernel(per_atom_property, atomic_subsystem_indices):
    n_atoms = per_atom_property.shape[0]
    idx32 = atomic_subsystem_indices.astype(jnp.int32)
    ncs_pad = ((n_atoms // CS + 15) // 16) * 16

    mesh = plsc.VectorSubcoreMesh(core_axis_name="c", subcore_axis_name="s")
    fn = pl.kernel(
        functools.partial(_sc_body, n_atoms),
        mesh=mesh,
        compiler_params=pltpu.CompilerParams(needs_layout_passes=False),
        out_type=jax.ShapeDtypeStruct((NUM_MOL,), jnp.float32),
        scratch_types=[
            pltpu.VMEM((W + 16,), jnp.float32),
            pltpu.VMEM((W + 16,), jnp.int32),
            pltpu.VMEM((W + 16,), jnp.float32),
            pltpu.VMEM((W + 16,), jnp.int32),
            pltpu.VMEM((ACC_PAD,), jnp.float32),
            pltpu.VMEM((ncs_pad,), jnp.int32),
            pltpu.VMEM((ncs_pad,), jnp.int32),
            pltpu.SemaphoreType.DMA,
            pltpu.SemaphoreType.DMA,
        ],
    )
    return fn(per_atom_property, idx32)


# speedup vs baseline: 1.3686x; 1.3686x over previous
import functools

import jax
import jax.numpy as jnp
from jax import lax
from jax.experimental import pallas as pl
from jax.experimental.pallas import tpu as pltpu
from jax.experimental.pallas import tpu_sc as plsc

NUM_MOL = 100000
NW = 32
MPW = 3136
MPW_LAST = NUM_MOL - (NW - 1) * MPW
ACC_PAD = MPW
W = 24576
WE = W - 16
CS = 2048


def _sc_body(n_atoms, vals_hbm, idx_hbm, out_hbm,
             vbufa, ibufa, vbufb, ibufb, acc, cilist, csbuf, sema, semb):
    c = lax.axis_index("c")
    s = lax.axis_index("s")
    w = s * 2 + c
    mw0 = w * MPW
    mpw_w = jnp.minimum(mw0 + MPW, NUM_MOL) - mw0

    iota0 = lax.iota(jnp.int32, 16)
    ncs = n_atoms // CS
    ncs_pad = ((ncs + 15) // 16) * 16

    off = w * 97

    def bld(i, _):
        slot = i * 16 + iota0
        pos = slot + off
        pos = jnp.where(pos >= ncs, pos - ncs, pos)
        pos = jnp.where(slot >= ncs, 0, pos)
        cilist[pl.ds(i * 16, 16)] = pos * CS
        return 0

    lax.fori_loop(0, ncs_pad // 16, bld, 0)
    pltpu.async_copy(idx_hbm.at[cilist], csbuf, sema).wait()

    t0v = jnp.full((16,), mw0, dtype=jnp.int32)
    t1v = jnp.full((16,), mw0 + mpw_w, dtype=jnp.int32)

    def cnt(i, carry):
        c0, c1 = carry
        slot = i * 16 + iota0
        valid = slot < ncs
        cs_v = csbuf[pl.ds(i * 16, 16)]
        c0 = c0 + plsc.all_reduce_population_count((cs_v < t0v) & valid)
        c1 = c1 + plsc.all_reduce_population_count((cs_v < t1v) & valid)
        return c0, c1

    zc = jnp.zeros((16,), jnp.int32)
    j0v, j1v = lax.fori_loop(0, ncs_pad // 16, cnt, (zc, zc))
    a0 = jnp.maximum(0, (j0v[0] - 1) * CS)
    a1 = jnp.minimum(n_atoms, j1v[0] * CS)

    sent = jnp.full((16,), NUM_MOL, dtype=jnp.int32)
    ibufa[pl.ds(W, 16)] = sent
    ibufb[pl.ds(W, 16)] = sent

    def zbody(i, _):
        acc[pl.ds(i * 16, 16)] = jnp.zeros((16,), jnp.float32)
        return 0

    lax.fori_loop(0, ACC_PAD // 16, zbody, 0)

    iota = lax.iota(jnp.int32, 16)
    m_end15 = iota == 15
    m_low15 = iota < 15
    mpw_u = jnp.full((16,), mpw_w, dtype=jnp.int32)

    nwin = jnp.maximum((a1 - a0 + WE - 1) // WE, 1)
    nfull = nwin - 1

    def issue(kk, vbuf, ibuf, sem):
        p0 = a0 + kk * WE
        st = jnp.minimum(p0, n_atoms - W)
        st = pl.multiple_of(st, 16)
        pltpu.async_copy(vals_hbm.at[pl.ds(st, W)], vbuf.at[pl.ds(0, W)], sem)
        pltpu.async_copy(idx_hbm.at[pl.ds(st, W)], ibuf.at[pl.ds(0, W)], sem)

    def wait(vbuf, ibuf, sem):
        pltpu.make_async_copy(vals_hbm.at[pl.ds(0, W)],
                              vbuf.at[pl.ds(0, W)], sem).wait()
        pltpu.make_async_copy(idx_hbm.at[pl.ds(0, W)],
                              ibuf.at[pl.ds(0, W)], sem).wait()

    issue(0, vbufa, ibufa, sema)

    def body_at(vbuf, ibuf, base, j):
        o = base + j * 16
        v = vbuf[pl.ds(o, 16)]
        ic = ibuf[pl.ds(o, 16)]
        inx = ibuf[pl.ds(o + 1, 16)]
        cum = plsc.cumsum(v)
        li = ic - mw0
        ln = inx - mw0
        chg = ic != inx
        in_i = plsc.bitcast(li, jnp.uint32) < plsc.bitcast(mpw_u, jnp.uint32)
        in_n = plsc.bitcast(ln, jnp.uint32) < plsc.bitcast(mpw_u, jnp.uint32)
        mend = (chg | m_end15) & in_i
        msub = chg & m_low15 & in_n
        plsc.addupdate_scatter(acc, [li], cum, mask=mend)
        plsc.addupdate_scatter(acc, [ln], -cum, mask=msub)

    def lean_loop(vbuf, ibuf):
        @plsc.parallel_loop(0, WE // 16, unroll=8)
        def _(j):
            body_at(vbuf, ibuf, 0, j)

    def lean_w(k, _unused):
        @pl.when(k % 2 == 0)
        def _():
            wait(vbufa, ibufa, sema)
            issue(k + 1, vbufb, ibufb, semb)
            lean_loop(vbufa, ibufa)

        @pl.when(k % 2 == 1)
        def _():
            wait(vbufb, ibufb, semb)
            issue(k + 1, vbufa, ibufa, sema)
            lean_loop(vbufb, ibufb)

        return 0

    lax.fori_loop(0, nfull, lean_w, 0)

    p0t = a0 + nfull * WE
    stt = jnp.minimum(p0t, n_atoms - W)
    base = pl.multiple_of(p0t - stt, 16)
    nblk = (a1 - p0t + 15) // 16

    def tail_loop(vbuf, ibuf):
        @plsc.parallel_loop(0, nblk, unroll=4)
        def _(j):
            body_at(vbuf, ibuf, base, j)

    @pl.when(nfull % 2 == 0)
    def _():
        wait(vbufa, ibufa, sema)
        tail_loop(vbufa, ibufa)

    @pl.when(nfull % 2 == 1)
    def _():
        wait(vbufb, ibufb, semb)
        tail_loop(vbufb, ibufb)

    @pl.when(w < NW - 1)
    def _():
        pltpu.sync_copy(acc.at[pl.ds(0, MPW)], out_hbm.at[pl.ds(mw0, MPW)])

    @pl.when(w == NW - 1)
    def _():
        pltpu.sync_copy(acc.at[pl.ds(0, MPW_LAST)],
                        out_hbm.at[pl.ds(mw0, MPW_LAST)])


@jax.jit
def kernel(per_atom_property, atomic_subsystem_indices):
    n_atoms = per_atom_property.shape[0]
    idx32 = atomic_subsystem_indices.astype(jnp.int32)
    ncs_pad = ((n_atoms // CS + 15) // 16) * 16

    mesh = plsc.VectorSubcoreMesh(core_axis_name="c", subcore_axis_name="s")
    fn = pl.kernel(
        functools.partial(_sc_body, n_atoms),
        mesh=mesh,
        compiler_params=pltpu.CompilerParams(needs_layout_passes=False),
        out_type=jax.ShapeDtypeStruct((NUM_MOL,), jnp.float32),
        scratch_types=[
            pltpu.VMEM((W + 16,), jnp.float32),
            pltpu.VMEM((W + 16,), jnp.int32),
            pltpu.VMEM((W + 16,), jnp.float32),
            pltpu.VMEM((W + 16,), jnp.int32),
            pltpu.VMEM((ACC_PAD,), jnp.float32),
            pltpu.VMEM((ncs_pad,), jnp.int32),
            pltpu.VMEM((ncs_pad,), jnp.int32),
            pltpu.SemaphoreType.DMA,
            pltpu.SemaphoreType.DMA,
        ],
    )
    return fn(per_atom_property, idx32)


# speedup vs baseline: 1.4454x; 1.0562x over previous
import functools

import jax
import jax.numpy as jnp
from jax import lax
from jax.experimental import pallas as pl
from jax.experimental.pallas import tpu as pltpu
from jax.experimental.pallas import tpu_sc as plsc

NUM_MOL = 100000
NW = 32
MPW = 3136
MPW_LAST = NUM_MOL - (NW - 1) * MPW
ACC_PAD = MPW
W = 16384
WE = W - 16
CS = 2048


def _sc_body(n_atoms, vals_hbm, idx_hbm, out_hbm,
             vbufa, ibufa, vbufb, ibufb, acc, cil1, csb1, cil2, csb2,
             sema, semb):
    c = lax.axis_index("c")
    s = lax.axis_index("s")
    w = s * 2 + c
    mw0 = w * MPW
    mpw_w = jnp.minimum(mw0 + MPW, NUM_MOL) - mw0

    iota0 = lax.iota(jnp.int32, 16)
    ncs = n_atoms // CS
    ncc = (ncs + 63) // 64

    t0v = jnp.full((16,), mw0, dtype=jnp.int32)
    t1v = jnp.full((16,), mw0 + mpw_w, dtype=jnp.int32)

    for i in range(4):
        slot = i * 16 + iota0
        cil1[pl.ds(i * 16, 16)] = jnp.where(slot < ncc, slot * (64 * CS), 0)
    pltpu.async_copy(idx_hbm.at[cil1], csb1, sema).wait()

    zc = jnp.zeros((16,), jnp.int32)
    c0 = c1 = zc
    for i in range(4):
        slot = i * 16 + iota0
        valid = slot < ncc
        cs_v = csb1[pl.ds(i * 16, 16)]
        c0 = c0 + plsc.all_reduce_population_count((cs_v < t0v) & valid)
        c1 = c1 + plsc.all_reduce_population_count((cs_v < t1v) & valid)
    b0 = jnp.maximum(c0[0] - 1, 0) * 64
    b1 = jnp.maximum(c1[0] - 1, 0) * 64

    for i in range(4):
        slot = i * 16 + iota0
        g0 = b0 + slot
        g1 = b1 + slot
        cil2[pl.ds(i * 16, 16)] = jnp.where(g0 < ncs, g0 * CS, 0)
        cil2[pl.ds(64 + i * 16, 16)] = jnp.where(g1 < ncs, g1 * CS, 0)
    pltpu.async_copy(idx_hbm.at[cil2], csb2, semb).wait()

    c0 = c1 = zc
    for i in range(4):
        slot = i * 16 + iota0
        cs0 = csb2[pl.ds(i * 16, 16)]
        cs1 = csb2[pl.ds(64 + i * 16, 16)]
        c0 = c0 + plsc.all_reduce_population_count(
            (cs0 < t0v) & (b0 + slot < ncs))
        c1 = c1 + plsc.all_reduce_population_count(
            (cs1 < t1v) & (b1 + slot < ncs))
    j0 = b0 + c0[0]
    j1 = b1 + c1[0]
    a0 = jnp.maximum(0, (j0 - 1) * CS)
    a1 = jnp.minimum(n_atoms, j1 * CS)

    sent = jnp.full((16,), NUM_MOL, dtype=jnp.int32)
    ibufa[pl.ds(W, 16)] = sent
    ibufb[pl.ds(W, 16)] = sent

    def zbody(i, _):
        acc[pl.ds(i * 16, 16)] = jnp.zeros((16,), jnp.float32)
        return 0

    lax.fori_loop(0, ACC_PAD // 16, zbody, 0)

    iota = lax.iota(jnp.int32, 16)
    m_end15 = iota == 15
    m_low15 = iota < 15
    mpw_u = jnp.full((16,), mpw_w, dtype=jnp.int32)

    nwin = jnp.maximum((a1 - a0 + WE - 1) // WE, 1)
    nfull = nwin - 1

    def issue(kk, vbuf, ibuf, sem):
        p0 = a0 + kk * WE
        st = jnp.minimum(p0, n_atoms - W)
        st = pl.multiple_of(st, 16)
        pltpu.async_copy(vals_hbm.at[pl.ds(st, W)], vbuf.at[pl.ds(0, W)], sem)
        pltpu.async_copy(idx_hbm.at[pl.ds(st, W)], ibuf.at[pl.ds(0, W)], sem)

    def wait(vbuf, ibuf, sem):
        pltpu.make_async_copy(vals_hbm.at[pl.ds(0, W)],
                              vbuf.at[pl.ds(0, W)], sem).wait()
        pltpu.make_async_copy(idx_hbm.at[pl.ds(0, W)],
                              ibuf.at[pl.ds(0, W)], sem).wait()

    issue(0, vbufa, ibufa, sema)

    def body_at(vbuf, ibuf, base, j):
        o = base + j * 16
        v = vbuf[pl.ds(o, 16)]
        ic = ibuf[pl.ds(o, 16)]
        inx = ibuf[pl.ds(o + 1, 16)]
        cum = plsc.cumsum(v)
        li = ic - mw0
        ln = inx - mw0
        chg = ic != inx
        in_i = plsc.bitcast(li, jnp.uint32) < plsc.bitcast(mpw_u, jnp.uint32)
        in_n = plsc.bitcast(ln, jnp.uint32) < plsc.bitcast(mpw_u, jnp.uint32)
        mend = (chg | m_end15) & in_i
        msub = chg & m_low15 & in_n
        plsc.addupdate_scatter(acc, [li], cum, mask=mend)
        plsc.addupdate_scatter(acc, [ln], -cum, mask=msub)

    def lean_loop(vbuf, ibuf):
        @plsc.parallel_loop(0, WE // 16, unroll=8)
        def _(j):
            body_at(vbuf, ibuf, 0, j)

    def lean_w(k, _unused):
        @pl.when(k % 2 == 0)
        def _():
            wait(vbufa, ibufa, sema)
            issue(k + 1, vbufb, ibufb, semb)
            lean_loop(vbufa, ibufa)

        @pl.when(k % 2 == 1)
        def _():
            wait(vbufb, ibufb, semb)
            issue(k + 1, vbufa, ibufa, sema)
            lean_loop(vbufb, ibufb)

        return 0

    lax.fori_loop(0, nfull, lean_w, 0)

    p0t = a0 + nfull * WE
    stt = jnp.minimum(p0t, n_atoms - W)
    base = pl.multiple_of(p0t - stt, 16)
    nblk = (a1 - p0t + 15) // 16

    def tail_loop(vbuf, ibuf):
        @plsc.parallel_loop(0, nblk, unroll=4)
        def _(j):
            body_at(vbuf, ibuf, base, j)

    @pl.when(nfull % 2 == 0)
    def _():
        wait(vbufa, ibufa, sema)
        tail_loop(vbufa, ibufa)

    @pl.when(nfull % 2 == 1)
    def _():
        wait(vbufb, ibufb, semb)
        tail_loop(vbufb, ibufb)

    @pl.when(w < NW - 1)
    def _():
        pltpu.sync_copy(acc.at[pl.ds(0, MPW)], out_hbm.at[pl.ds(mw0, MPW)])

    @pl.when(w == NW - 1)
    def _():
        pltpu.sync_copy(acc.at[pl.ds(0, MPW_LAST)],
                        out_hbm.at[pl.ds(mw0, MPW_LAST)])


@jax.jit
def kernel(per_atom_property, atomic_subsystem_indices):
    n_atoms = per_atom_property.shape[0]
    idx32 = atomic_subsystem_indices.astype(jnp.int32)

    mesh = plsc.VectorSubcoreMesh(core_axis_name="c", subcore_axis_name="s")
    fn = pl.kernel(
        functools.partial(_sc_body, n_atoms),
        mesh=mesh,
        compiler_params=pltpu.CompilerParams(needs_layout_passes=False),
        out_type=jax.ShapeDtypeStruct((NUM_MOL,), jnp.float32),
        scratch_types=[
            pltpu.VMEM((W + 16,), jnp.float32),
            pltpu.VMEM((W + 16,), jnp.int32),
            pltpu.VMEM((W + 16,), jnp.float32),
            pltpu.VMEM((W + 16,), jnp.int32),
            pltpu.VMEM((ACC_PAD,), jnp.float32),
            pltpu.VMEM((64,), jnp.int32),
            pltpu.VMEM((64,), jnp.int32),
            pltpu.VMEM((128,), jnp.int32),
            pltpu.VMEM((128,), jnp.int32),
            pltpu.SemaphoreType.DMA,
            pltpu.SemaphoreType.DMA,
        ],
    )
    return fn(per_atom_property, idx32)


# speedup vs baseline: 1.4527x; 1.0051x over previous
import functools

import jax
import jax.numpy as jnp
from jax import lax
from jax.experimental import pallas as pl
from jax.experimental.pallas import tpu as pltpu
from jax.experimental.pallas import tpu_sc as plsc

NUM_MOL = 100000
NW = 32
MPW = 3136
MPW_LAST = NUM_MOL - (NW - 1) * MPW
ACC_PAD = MPW
W = 16384
WE = W - 16
CS = 2048


def _sc_body(n_atoms, vals_hbm, idx_hbm, out_hbm,
             vbufa, ibufa, vbufb, ibufb, acc, cil1, csb1, cil2, csb2,
             sema, semb):
    c = lax.axis_index("c")
    s = lax.axis_index("s")
    w = s * 2 + c
    mw0 = w * MPW
    mpw_w = jnp.minimum(mw0 + MPW, NUM_MOL) - mw0

    iota0 = lax.iota(jnp.int32, 16)
    ncs = n_atoms // CS
    ncc = (ncs + 63) // 64

    t0v = jnp.full((16,), mw0, dtype=jnp.int32)
    t1v = jnp.full((16,), mw0 + mpw_w, dtype=jnp.int32)

    for i in range(4):
        slot = i * 16 + iota0
        cil1[pl.ds(i * 16, 16)] = jnp.where(slot < ncc, slot * (64 * CS), 0)
    pltpu.async_copy(idx_hbm.at[cil1], csb1, sema).wait()

    zc = jnp.zeros((16,), jnp.int32)
    c0 = c1 = zc
    for i in range(4):
        slot = i * 16 + iota0
        valid = slot < ncc
        cs_v = csb1[pl.ds(i * 16, 16)]
        c0 = c0 + plsc.all_reduce_population_count((cs_v < t0v) & valid)
        c1 = c1 + plsc.all_reduce_population_count((cs_v < t1v) & valid)
    b0 = jnp.maximum(c0[0] - 1, 0) * 64
    b1 = jnp.maximum(c1[0] - 1, 0) * 64

    for i in range(4):
        slot = i * 16 + iota0
        g0 = b0 + slot
        g1 = b1 + slot
        cil2[pl.ds(i * 16, 16)] = jnp.where(g0 < ncs, g0 * CS, 0)
        cil2[pl.ds(64 + i * 16, 16)] = jnp.where(g1 < ncs, g1 * CS, 0)
    pltpu.async_copy(idx_hbm.at[cil2], csb2, semb).wait()

    c0 = c1 = zc
    for i in range(4):
        slot = i * 16 + iota0
        cs0 = csb2[pl.ds(i * 16, 16)]
        cs1 = csb2[pl.ds(64 + i * 16, 16)]
        c0 = c0 + plsc.all_reduce_population_count(
            (cs0 < t0v) & (b0 + slot < ncs))
        c1 = c1 + plsc.all_reduce_population_count(
            (cs1 < t1v) & (b1 + slot < ncs))
    j0 = b0 + c0[0]
    j1 = b1 + c1[0]
    a0 = jnp.maximum(0, (j0 - 1) * CS)
    a1 = jnp.minimum(n_atoms, j1 * CS)

    sent = jnp.full((16,), NUM_MOL, dtype=jnp.int32)
    ibufa[pl.ds(W, 16)] = sent
    ibufb[pl.ds(W, 16)] = sent

    @plsc.parallel_loop(0, ACC_PAD // 16, unroll=8)
    def _(i):
        acc[pl.ds(i * 16, 16)] = jnp.zeros((16,), jnp.float32)

    iota = lax.iota(jnp.int32, 16)
    m_end15 = iota == 15
    m_low15 = iota < 15
    mpw_u = jnp.full((16,), mpw_w, dtype=jnp.int32)

    nwin = jnp.maximum((a1 - a0 + WE - 1) // WE, 1)
    nfull = nwin - 1

    def issue(kk, vbuf, ibuf, sem):
        p0 = a0 + kk * WE
        st = jnp.minimum(p0, n_atoms - W)
        st = pl.multiple_of(st, 16)
        pltpu.async_copy(vals_hbm.at[pl.ds(st, W)], vbuf.at[pl.ds(0, W)], sem)
        pltpu.async_copy(idx_hbm.at[pl.ds(st, W)], ibuf.at[pl.ds(0, W)], sem)

    def wait(vbuf, ibuf, sem):
        pltpu.make_async_copy(vals_hbm.at[pl.ds(0, W)],
                              vbuf.at[pl.ds(0, W)], sem).wait()
        pltpu.make_async_copy(idx_hbm.at[pl.ds(0, W)],
                              ibuf.at[pl.ds(0, W)], sem).wait()

    issue(0, vbufa, ibufa, sema)

    def body_at(vbuf, ibuf, base, j):
        o = base + j * 16
        v = vbuf[pl.ds(o, 16)]
        ic = ibuf[pl.ds(o, 16)]
        inx = ibuf[pl.ds(o + 1, 16)]
        cum = plsc.cumsum(v)
        li = ic - mw0
        ln = inx - mw0
        chg = ic != inx
        in_i = plsc.bitcast(li, jnp.uint32) < plsc.bitcast(mpw_u, jnp.uint32)
        in_n = plsc.bitcast(ln, jnp.uint32) < plsc.bitcast(mpw_u, jnp.uint32)
        mend = (chg | m_end15) & in_i
        msub = chg & m_low15 & in_n
        plsc.addupdate_scatter(acc, [li], cum, mask=mend)
        plsc.addupdate_scatter(acc, [ln], -cum, mask=msub)

    def mid_at(vbuf, ibuf, j):
        o = pl.multiple_of(j * 16, 16)
        v = vbuf[pl.ds(o, 16)]
        ic = ibuf[pl.ds(o, 16)]
        inx = ibuf[pl.ds(o + 1, 16)]
        cum = plsc.cumsum(v)
        chg = ic != inx
        mend = chg | m_end15
        msub = chg & m_low15
        plsc.addupdate_scatter(acc, [ic - mw0], cum, mask=mend)
        plsc.addupdate_scatter(acc, [inx - mw0], -cum, mask=msub)

    def lean_loop(vbuf, ibuf, is_mid):
        @pl.when(is_mid)
        def _():
            @plsc.parallel_loop(0, WE // 16, unroll=8)
            def _(j):
                mid_at(vbuf, ibuf, j)

        @pl.when(jnp.logical_not(is_mid))
        def _():
            @plsc.parallel_loop(0, WE // 16, unroll=8)
            def _(j):
                body_at(vbuf, ibuf, 0, j)

    def lean_w(k, _unused):
        is_mid = (k > 0) & (a0 + (k + 1) * WE <= a1 - CS)

        @pl.when(k % 2 == 0)
        def _():
            wait(vbufa, ibufa, sema)
            issue(k + 1, vbufb, ibufb, semb)
            lean_loop(vbufa, ibufa, is_mid)

        @pl.when(k % 2 == 1)
        def _():
            wait(vbufb, ibufb, semb)
            issue(k + 1, vbufa, ibufa, sema)
            lean_loop(vbufb, ibufb, is_mid)

        return 0

    lax.fori_loop(0, nfull, lean_w, 0)

    p0t = a0 + nfull * WE
    stt = jnp.minimum(p0t, n_atoms - W)
    base = pl.multiple_of(p0t - stt, 16)
    nblk = (a1 - p0t + 15) // 16

    def tail_loop(vbuf, ibuf):
        @plsc.parallel_loop(0, nblk, unroll=4)
        def _(j):
            body_at(vbuf, ibuf, base, j)

    @pl.when(nfull % 2 == 0)
    def _():
        wait(vbufa, ibufa, sema)
        tail_loop(vbufa, ibufa)

    @pl.when(nfull % 2 == 1)
    def _():
        wait(vbufb, ibufb, semb)
        tail_loop(vbufb, ibufb)

    @pl.when(w < NW - 1)
    def _():
        pltpu.sync_copy(acc.at[pl.ds(0, MPW)], out_hbm.at[pl.ds(mw0, MPW)])

    @pl.when(w == NW - 1)
    def _():
        pltpu.sync_copy(acc.at[pl.ds(0, MPW_LAST)],
                        out_hbm.at[pl.ds(mw0, MPW_LAST)])


@jax.jit
def kernel(per_atom_property, atomic_subsystem_indices):
    n_atoms = per_atom_property.shape[0]
    idx32 = atomic_subsystem_indices.astype(jnp.int32)

    mesh = plsc.VectorSubcoreMesh(core_axis_name="c", subcore_axis_name="s")
    fn = pl.kernel(
        functools.partial(_sc_body, n_atoms),
        mesh=mesh,
        compiler_params=pltpu.CompilerParams(needs_layout_passes=False),
        out_type=jax.ShapeDtypeStruct((NUM_MOL,), jnp.float32),
        scratch_types=[
            pltpu.VMEM((W + 16,), jnp.float32),
            pltpu.VMEM((W + 16,), jnp.int32),
            pltpu.VMEM((W + 16,), jnp.float32),
            pltpu.VMEM((W + 16,), jnp.int32),
            pltpu.VMEM((ACC_PAD,), jnp.float32),
            pltpu.VMEM((64,), jnp.int32),
            pltpu.VMEM((64,), jnp.int32),
            pltpu.VMEM((128,), jnp.int32),
            pltpu.VMEM((128,), jnp.int32),
            pltpu.SemaphoreType.DMA,
            pltpu.SemaphoreType.DMA,
        ],
    )
    return fn(per_atom_property, idx32)
